# Initial kernel scaffold; baseline (speedup 1.0000x reference)
import jax
import jax.numpy as jnp
from jax.experimental import pallas as pl


def kernel(x, weight, bias, gamma, beta):
    raise NotImplementedError("write your pallas kernel here")


# speedup vs baseline: 17.8059x; 17.8059x over previous
import jax
import jax.numpy as jnp
from jax.experimental import pallas as pl
from jax.experimental.pallas import tpu as pltpu

_N = 32
_CIN = 64
_HO = 56
_WO = 56
_COUT = 128
_HOWO = _HO * _WO
_ROWS = (_HO + 2) * _WO
_KDIM = 3 * _CIN
_CHUNK = 448
_NCHUNK = _HOWO // _CHUNK


def _conv_stats_kernel(x_ref, w_ref, o_ref, s_ref, ss_ref):
    s_tot = jnp.zeros((1, _COUT), jnp.float32)
    ss_tot = jnp.zeros((1, _COUT), jnp.float32)
    for c in range(_NCHUNK):
        m0 = c * _CHUNK
        a0 = x_ref[0, m0:m0 + _CHUNK, :]
        a1 = x_ref[0, m0 + _WO:m0 + _WO + _CHUNK, :]
        a2 = x_ref[0, m0 + 2 * _WO:m0 + 2 * _WO + _CHUNK, :]
        acc = (jnp.dot(a0, w_ref[0], preferred_element_type=jnp.float32)
               + jnp.dot(a1, w_ref[1], preferred_element_type=jnp.float32)
               + jnp.dot(a2, w_ref[2], preferred_element_type=jnp.float32))
        o_ref[0, m0:m0 + _CHUNK, :] = acc.astype(jnp.bfloat16)
        s_tot = s_tot + jnp.sum(acc, axis=0, keepdims=True)
        ss_tot = ss_tot + jnp.sum(acc * acc, axis=0, keepdims=True)
    s_ref[0] = s_tot
    ss_ref[0] = ss_tot


def _bn_lrelu_t_kernel(x_ref, sc_ref, sh_ref, o_ref):
    for c in range(_NCHUNK):
        m0 = c * _CHUNK
        y = (x_ref[0, m0:m0 + _CHUNK, :].astype(jnp.float32)
             * sc_ref[...] + sh_ref[...])
        y = jnp.where(y > 0, y, jnp.float32(0.01) * y)
        o_ref[0, :, m0:m0 + _CHUNK] = y.T


def kernel(x, weight, bias, gamma, beta):
    del bias

    xt = jnp.transpose(x, (0, 2, 3, 1)).astype(jnp.bfloat16)
    xp = jnp.pad(xt, ((0, 0), (1, 1), (1, 1), (0, 0)))
    xcat = jnp.concatenate(
        [xp[:, :, 0:_WO, :], xp[:, :, 1:_WO + 1, :], xp[:, :, 2:_WO + 2, :]],
        axis=-1)
    xflat = xcat.reshape(_N, _ROWS, _KDIM)
    wcat = jnp.transpose(weight, (2, 3, 1, 0)).reshape(3, _KDIM, _COUT)
    wcat = wcat.astype(jnp.bfloat16)

    conv, s, ss = pl.pallas_call(
        _conv_stats_kernel,
        out_shape=(jax.ShapeDtypeStruct((_N, _HOWO, _COUT), jnp.bfloat16),
                   jax.ShapeDtypeStruct((_N, 1, _COUT), jnp.float32),
                   jax.ShapeDtypeStruct((_N, 1, _COUT), jnp.float32)),
        grid=(_N,),
        in_specs=[pl.BlockSpec((1, _ROWS, _KDIM), lambda i: (i, 0, 0)),
                  pl.BlockSpec((3, _KDIM, _COUT), lambda i: (0, 0, 0))],
        out_specs=[pl.BlockSpec((1, _HOWO, _COUT), lambda i: (i, 0, 0)),
                   pl.BlockSpec((1, 1, _COUT), lambda i: (i, 0, 0)),
                   pl.BlockSpec((1, 1, _COUT), lambda i: (i, 0, 0))],
        compiler_params=pltpu.CompilerParams(
            dimension_semantics=("parallel",),
            vmem_limit_bytes=32 * 1024 * 1024,
        ),
    )(xflat, wcat)

    count = jnp.float32(_N * _HOWO)
    sums = jnp.sum(s[:, 0, :], axis=0)
    sumsq = jnp.sum(ss[:, 0, :], axis=0)
    mean = sums / count
    var = jnp.maximum(sumsq / count - mean * mean, 0.0)
    scale = gamma / jnp.sqrt(var + jnp.float32(1e-5))
    shift = beta - scale * mean

    y = pl.pallas_call(
        _bn_lrelu_t_kernel,
        out_shape=jax.ShapeDtypeStruct((_N, _COUT, _HOWO), jnp.float32),
        grid=(_N,),
        in_specs=[pl.BlockSpec((1, _HOWO, _COUT), lambda b: (b, 0, 0)),
                  pl.BlockSpec((1, _COUT), lambda b: (0, 0)),
                  pl.BlockSpec((1, _COUT), lambda b: (0, 0))],
        out_specs=pl.BlockSpec((1, _COUT, _HOWO), lambda b: (b, 0, 0)),
        compiler_params=pltpu.CompilerParams(
            dimension_semantics=("parallel",),
            vmem_limit_bytes=32 * 1024 * 1024,
        ),
    )(conv, scale.reshape(1, _COUT), shift.reshape(1, _COUT))

    return y.reshape(_N, _COUT, _HO, _WO)


# speedup vs baseline: 23.0242x; 1.2931x over previous
import jax
import jax.numpy as jnp
from jax.experimental import pallas as pl
from jax.experimental.pallas import tpu as pltpu

_N = 32
_CIN = 64
_HO = 56
_WO = 56
_COUT = 128
_HOWO = _HO * _WO
_ROWS = (_HO + 2) * _WO
_KDIM = 3 * _CIN
_CHUNK = 448
_NCHUNK = _HOWO // _CHUNK


def _conv_stats_kernel(x_ref, w_ref, mw_ref, o_ref, s_ref, ss_ref, xf_ref):
    xt = x_ref[0].T.astype(jnp.bfloat16)
    zeros_row = jnp.zeros((_WO, _KDIM), jnp.bfloat16)
    xf_ref[0:_WO, :] = zeros_row
    xf_ref[_ROWS - _WO:_ROWS, :] = zeros_row
    xf_ref[_WO:_ROWS - _WO, _CIN:2 * _CIN] = xt
    v0 = xt * mw_ref[:, 0:_CIN]
    xf_ref[_WO + 1:_ROWS - _WO, 0:_CIN] = v0[0:_HOWO - 1, :]
    xf_ref[_WO:_WO + 1, 0:_CIN] = jnp.zeros((1, _CIN), jnp.bfloat16)
    v2 = xt * mw_ref[:, _CIN:2 * _CIN]
    xf_ref[_WO:_ROWS - _WO - 1, 2 * _CIN:] = v2[1:_HOWO, :]
    xf_ref[_ROWS - _WO - 1:_ROWS - _WO, 2 * _CIN:] = (
        jnp.zeros((1, _CIN), jnp.bfloat16))

    s_tot = jnp.zeros((1, _COUT), jnp.float32)
    ss_tot = jnp.zeros((1, _COUT), jnp.float32)
    for c in range(_NCHUNK):
        m0 = c * _CHUNK
        a0 = xf_ref[m0:m0 + _CHUNK, :]
        a1 = xf_ref[m0 + _WO:m0 + _WO + _CHUNK, :]
        a2 = xf_ref[m0 + 2 * _WO:m0 + 2 * _WO + _CHUNK, :]
        acc = (jnp.dot(a0, w_ref[0], preferred_element_type=jnp.float32)
               + jnp.dot(a1, w_ref[1], preferred_element_type=jnp.float32)
               + jnp.dot(a2, w_ref[2], preferred_element_type=jnp.float32))
        o_ref[0, m0:m0 + _CHUNK, :] = acc.astype(jnp.bfloat16)
        s_tot = s_tot + jnp.sum(acc, axis=0, keepdims=True)
        ss_tot = ss_tot + jnp.sum(acc * acc, axis=0, keepdims=True)
    s_ref[0] = s_tot
    ss_ref[0] = ss_tot


def _bn_lrelu_t_kernel(x_ref, sc_ref, sh_ref, o_ref):
    for c in range(_NCHUNK):
        m0 = c * _CHUNK
        y = (x_ref[0, m0:m0 + _CHUNK, :].astype(jnp.float32)
             * sc_ref[...] + sh_ref[...])
        y = jnp.where(y > 0, y, jnp.float32(0.01) * y)
        o_ref[0, :, m0:m0 + _CHUNK] = y.T


def kernel(x, weight, bias, gamma, beta):
    del bias

    x3 = x.reshape(_N, _CIN, _HOWO)
    wcat = jnp.transpose(weight, (2, 3, 1, 0)).reshape(3, _KDIM, _COUT)
    wcat = wcat.astype(jnp.bfloat16)
    ridx = jnp.arange(_HOWO) % _WO
    mw = jnp.concatenate(
        [jnp.broadcast_to((ridx != _WO - 1)[:, None], (_HOWO, _CIN)),
         jnp.broadcast_to((ridx != 0)[:, None], (_HOWO, _CIN))],
        axis=1).astype(jnp.bfloat16)

    conv, s, ss = pl.pallas_call(
        _conv_stats_kernel,
        out_shape=(jax.ShapeDtypeStruct((_N, _HOWO, _COUT), jnp.bfloat16),
                   jax.ShapeDtypeStruct((_N, 1, _COUT), jnp.float32),
                   jax.ShapeDtypeStruct((_N, 1, _COUT), jnp.float32)),
        grid=(_N,),
        in_specs=[pl.BlockSpec((1, _CIN, _HOWO), lambda i: (i, 0, 0)),
                  pl.BlockSpec((3, _KDIM, _COUT), lambda i: (0, 0, 0)),
                  pl.BlockSpec((_HOWO, 2 * _CIN), lambda i: (0, 0))],
        out_specs=[pl.BlockSpec((1, _HOWO, _COUT), lambda i: (i, 0, 0)),
                   pl.BlockSpec((1, 1, _COUT), lambda i: (i, 0, 0)),
                   pl.BlockSpec((1, 1, _COUT), lambda i: (i, 0, 0))],
        scratch_shapes=[pltpu.VMEM((_ROWS, _KDIM), jnp.bfloat16)],
        compiler_params=pltpu.CompilerParams(
            dimension_semantics=("parallel",),
            vmem_limit_bytes=32 * 1024 * 1024,
        ),
    )(x3, wcat, mw)

    count = jnp.float32(_N * _HOWO)
    sums = jnp.sum(s[:, 0, :], axis=0)
    sumsq = jnp.sum(ss[:, 0, :], axis=0)
    mean = sums / count
    var = jnp.maximum(sumsq / count - mean * mean, 0.0)
    scale = gamma / jnp.sqrt(var + jnp.float32(1e-5))
    shift = beta - scale * mean

    y = pl.pallas_call(
        _bn_lrelu_t_kernel,
        out_shape=jax.ShapeDtypeStruct((_N, _COUT, _HOWO), jnp.float32),
        grid=(_N,),
        in_specs=[pl.BlockSpec((1, _HOWO, _COUT), lambda b: (b, 0, 0)),
                  pl.BlockSpec((1, _COUT), lambda b: (0, 0)),
                  pl.BlockSpec((1, _COUT), lambda b: (0, 0))],
        out_specs=pl.BlockSpec((1, _COUT, _HOWO), lambda b: (b, 0, 0)),
        compiler_params=pltpu.CompilerParams(
            dimension_semantics=("parallel",),
            vmem_limit_bytes=32 * 1024 * 1024,
        ),
    )(conv, scale.reshape(1, _COUT), shift.reshape(1, _COUT))

    return y.reshape(_N, _COUT, _HO, _WO)
